# baseline (speedup 1.0000x reference)
import functools

import jax
import jax.numpy as jnp
import numpy as np
from jax import lax
from jax.experimental import pallas as pl
from jax.experimental.pallas import tpu as pltpu
from jax.experimental.pallas import tpu_sc as plsc

N_ROWS = 65536
N_CH = 256
N_GROUPS = 8
GROUP = 32
EPS = 1e-5

NUM_CORES = 2
NUM_SUBCORES = 16
NW = NUM_CORES * NUM_SUBCORES
ROWS_PER_W = N_ROWS // NW
CHUNK = 64
N_CHUNKS = ROWS_PER_W // CHUNK

_RSQRT_MAGIC = np.int32(0x5F3759DF)


def _rsqrt(v):
    i = lax.bitcast_convert_type(v, jnp.int32)
    y = lax.bitcast_convert_type(_RSQRT_MAGIC - lax.shift_right_logical(i, 1),
                                 jnp.float32)
    half = jnp.float32(0.5) * v
    for _ in range(2):
        y = y * (jnp.float32(1.5) - half * y * y)
    return y


def _compute_chunk(in_v, out_v):
    lane = lax.broadcasted_iota(jnp.int32, (16,), 0)
    c15 = jnp.full((16,), 15, jnp.int32)
    masks = [lane == g for g in range(N_GROUPS)]
    zeros = jnp.zeros((16,), jnp.float32)

    @plsc.parallel_loop(0, CHUNK, unroll=2)
    def row_body(r):
        var_all = zeros
        ds = []
        for g in range(N_GROUPS):
            a = in_v[r, pl.ds(g * GROUP, 16)]
            b = in_v[r, pl.ds(g * GROUP + 16, 16)]
            t = a + b
            s_b = plsc.cumsum(t).at[c15].get(mode="promise_in_bounds")
            mean = s_b * jnp.float32(1.0 / GROUP)
            da = a - mean
            db = b - mean
            qt = da * da + db * db
            q_b = plsc.cumsum(qt).at[c15].get(mode="promise_in_bounds")
            var_all = jnp.where(masks[g], q_b, var_all)
            ds.append((da, db))
        var_all = var_all * jnp.float32(1.0 / (GROUP - 1))
        std = var_all * _rsqrt(var_all)
        rcp_all = jnp.ones((16,), jnp.float32) / (std + jnp.float32(EPS))
        for g in range(N_GROUPS):
            r_g = rcp_all.at[jnp.full((16,), g, jnp.int32)].get(
                mode="promise_in_bounds")
            out_v[pl.ds(r * N_CH + g * GROUP, 16)] = ds[g][0] * r_g
            out_v[pl.ds(r * N_CH + g * GROUP + 16, 16)] = ds[g][1] * r_g


def _body(x_hbm, out_hbm, in0, in1, ou0, ou1, isem0, isem1, osem0, osem1):
    cid = lax.axis_index("c")
    sid = lax.axis_index("s")
    wid = sid * NUM_CORES + cid
    base_row = wid * ROWS_PER_W

    ins = (in0, in1)
    ous = (ou0, ou1)
    isems = (isem0, isem1)
    osems = (osem0, osem1)

    def in_slice(ci):
        return x_hbm.at[pl.ds(base_row + ci * CHUNK, CHUNK)]

    def out_slice(ci):
        return out_hbm.at[pl.ds((base_row + ci * CHUNK) * N_CH, CHUNK * N_CH)]

    pltpu.async_copy(in_slice(0), ins[0], isems[0])

    n_pairs = N_CHUNKS // 2

    def pair_body(p, _):
        for b in range(2):
            ci = p * 2 + b
            nb = 1 - b

            @pl.when(ci + 1 < N_CHUNKS)
            def _():
                pltpu.async_copy(in_slice(ci + 1), ins[nb], isems[nb])

            pltpu.make_async_copy(in_slice(ci), ins[b], isems[b]).wait()

            @pl.when(ci >= 2)
            def _():
                pltpu.make_async_copy(ous[b], out_slice(ci), osems[b]).wait()

            _compute_chunk(ins[b], ous[b])
            pltpu.async_copy(ous[b], out_slice(ci), osems[b])
        return 0

    lax.fori_loop(0, n_pairs, pair_body, 0)

    pltpu.make_async_copy(ous[0], out_slice(N_CHUNKS - 2), osems[0]).wait()
    pltpu.make_async_copy(ous[1], out_slice(N_CHUNKS - 1), osems[1]).wait()


@jax.jit
def _run(x):
    mesh = plsc.VectorSubcoreMesh(core_axis_name="c", subcore_axis_name="s")
    f = pl.kernel(
        _body,
        out_type=jax.ShapeDtypeStruct((N_ROWS * N_CH,), jnp.float32),
        mesh=mesh,
        scratch_types=[
            pltpu.VMEM((CHUNK, N_CH), jnp.float32),
            pltpu.VMEM((CHUNK, N_CH), jnp.float32),
            pltpu.VMEM((CHUNK * N_CH,), jnp.float32),
            pltpu.VMEM((CHUNK * N_CH,), jnp.float32),
            pltpu.SemaphoreType.DMA,
            pltpu.SemaphoreType.DMA,
            pltpu.SemaphoreType.DMA,
            pltpu.SemaphoreType.DMA,
        ],
        compiler_params=pltpu.CompilerParams(needs_layout_passes=False,
                                             use_tc_tiling_on_sc=True),
    )
    return f(x)


def kernel(x, channel_groups):
    del channel_groups
    out = _run(x)
    return out.reshape(N_ROWS, N_CH, 1)


# speedup vs baseline: 1.0418x; 1.0418x over previous
import functools

import jax
import jax.numpy as jnp
import numpy as np
from jax import lax
from jax.experimental import pallas as pl
from jax.experimental.pallas import tpu as pltpu
from jax.experimental.pallas import tpu_sc as plsc

N_ROWS = 65536
N_CH = 256
N_GROUPS = 8
GROUP = 32
EPS = 1e-5

NUM_CORES = 2
NUM_SUBCORES = 16
NW = NUM_CORES * NUM_SUBCORES
ROWS_PER_W = N_ROWS // NW
CHUNK = 64
N_CHUNKS = ROWS_PER_W // CHUNK

_RSQRT_MAGIC = np.int32(0x5F3759DF)


def _rsqrt(v):
    i = lax.bitcast_convert_type(v, jnp.int32)
    y = lax.bitcast_convert_type(_RSQRT_MAGIC - lax.shift_right_logical(i, 1),
                                 jnp.float32)
    half = jnp.float32(0.5) * v
    for _ in range(2):
        y = y * (jnp.float32(1.5) - half * y * y)
    return y


def _compute_chunk(in_v, out_v):
    lane = lax.broadcasted_iota(jnp.int32, (16,), 0)
    c15 = jnp.full((16,), 15, jnp.int32)
    masks = [lane == g for g in range(N_GROUPS)]
    zeros = jnp.zeros((16,), jnp.float32)

    @plsc.parallel_loop(0, CHUNK)
    def row_body(r):
        var_all = zeros
        ds = []
        for g in range(N_GROUPS):
            a = in_v[r, pl.ds(g * GROUP, 16)]
            b = in_v[r, pl.ds(g * GROUP + 16, 16)]
            t = a + b
            s_b = plsc.cumsum(t).at[c15].get(mode="promise_in_bounds")
            mean = s_b * jnp.float32(1.0 / GROUP)
            da = a - mean
            db = b - mean
            qt = da * da + db * db
            q_b = plsc.cumsum(qt).at[c15].get(mode="promise_in_bounds")
            var_all = jnp.where(masks[g], q_b, var_all)
            ds.append((da, db))
        var_all = var_all * jnp.float32(1.0 / (GROUP - 1))
        std = var_all * _rsqrt(var_all)
        rcp_all = jnp.ones((16,), jnp.float32) / (std + jnp.float32(EPS))
        for g in range(N_GROUPS):
            r_g = rcp_all.at[jnp.full((16,), g, jnp.int32)].get(
                mode="promise_in_bounds")
            out_v[pl.ds(r * N_CH + g * GROUP, 16)] = ds[g][0] * r_g
            out_v[pl.ds(r * N_CH + g * GROUP + 16, 16)] = ds[g][1] * r_g


def _body(x_hbm, out_hbm, in0, in1, ou0, ou1, isem0, isem1, osem0, osem1):
    cid = lax.axis_index("c")
    sid = lax.axis_index("s")
    wid = sid * NUM_CORES + cid
    base_row = wid * ROWS_PER_W

    ins = (in0, in1)
    ous = (ou0, ou1)
    isems = (isem0, isem1)
    osems = (osem0, osem1)

    def in_slice(ci):
        return x_hbm.at[pl.ds(base_row + ci * CHUNK, CHUNK)]

    def out_slice(ci):
        return out_hbm.at[pl.ds((base_row + ci * CHUNK) * N_CH, CHUNK * N_CH)]

    pltpu.async_copy(in_slice(0), ins[0], isems[0])

    n_pairs = N_CHUNKS // 2

    def pair_body(p, _):
        for b in range(2):
            ci = p * 2 + b
            nb = 1 - b

            @pl.when(ci + 1 < N_CHUNKS)
            def _():
                pltpu.async_copy(in_slice(ci + 1), ins[nb], isems[nb])

            pltpu.make_async_copy(in_slice(ci), ins[b], isems[b]).wait()

            @pl.when(ci >= 2)
            def _():
                pltpu.make_async_copy(ous[b], out_slice(ci), osems[b]).wait()

            _compute_chunk(ins[b], ous[b])
            pltpu.async_copy(ous[b], out_slice(ci), osems[b])
        return 0

    lax.fori_loop(0, n_pairs, pair_body, 0)

    pltpu.make_async_copy(ous[0], out_slice(N_CHUNKS - 2), osems[0]).wait()
    pltpu.make_async_copy(ous[1], out_slice(N_CHUNKS - 1), osems[1]).wait()


@jax.jit
def _run(x):
    mesh = plsc.VectorSubcoreMesh(core_axis_name="c", subcore_axis_name="s")
    f = pl.kernel(
        _body,
        out_type=jax.ShapeDtypeStruct((N_ROWS * N_CH,), jnp.float32),
        mesh=mesh,
        scratch_types=[
            pltpu.VMEM((CHUNK, N_CH), jnp.float32),
            pltpu.VMEM((CHUNK, N_CH), jnp.float32),
            pltpu.VMEM((CHUNK * N_CH,), jnp.float32),
            pltpu.VMEM((CHUNK * N_CH,), jnp.float32),
            pltpu.SemaphoreType.DMA,
            pltpu.SemaphoreType.DMA,
            pltpu.SemaphoreType.DMA,
            pltpu.SemaphoreType.DMA,
        ],
        compiler_params=pltpu.CompilerParams(needs_layout_passes=False,
                                             use_tc_tiling_on_sc=True),
    )
    return f(x)


def kernel(x, channel_groups):
    del channel_groups
    out = _run(x)
    return out.reshape(N_ROWS, N_CH, 1)
